# baseline (speedup 1.0000x reference)
import functools

import jax
import jax.numpy as jnp
from jax import lax
from jax.experimental import pallas as pl
from jax.experimental.pallas import tpu as pltpu
from jax.experimental.pallas import tpu_sc as plsc

B = 4096
F = 64
NSETS = 4
IDX_W = 128
ROWS = NSETS * B // IDX_W
NC, NS = 2, 16
NW = NC * NS
RPW = ROWS // NW


def _sc_gather(table, idx2d):
    mesh = plsc.VectorSubcoreMesh(core_axis_name="c", subcore_axis_name="s")

    @functools.partial(
        pl.kernel,
        mesh=mesh,
        compiler_params=pltpu.CompilerParams(use_tc_tiling_on_sc=False),
        out_type=jax.ShapeDtypeStruct((ROWS, IDX_W, F), jnp.float32),
        scratch_types=[
            pltpu.VMEM((RPW, IDX_W), jnp.int32),
            pltpu.VMEM((RPW, IDX_W, F), jnp.float32),
            pltpu.SemaphoreType.DMA,
        ],
    )
    def k(table_hbm, idx_hbm, out_hbm, idx_v, rows_v, sem):
        wid = lax.axis_index("s") * NC + lax.axis_index("c")
        base = wid * RPW
        pltpu.sync_copy(idx_hbm.at[pl.ds(base, RPW)], idx_v)
        cps = [
            pltpu.async_copy(table_hbm.at[idx_v.at[r]], rows_v.at[r], sem)
            for r in range(RPW)
        ]
        for cp in cps:
            cp.wait()
        pltpu.sync_copy(rows_v, out_hbm.at[pl.ds(base, RPW)])

    return k(table, idx2d)


def _tc_body(rows_ref, w1_ref, b1_ref, w2_ref, b2_ref, w3_ref, c_ref,
             oi_ref, oj_ref):
    dot = functools.partial(jax.lax.dot,
                            precision=jax.lax.Precision.HIGHEST,
                            preferred_element_type=jnp.float32)
    eu = rows_ref[0]
    ei = rows_ref[1]
    ej = rows_ref[2]
    ec = rows_ref[3]
    w1 = w1_ref[...]
    shared = dot(eu, w1[0:F]) + dot(ec, w1[2 * F:3 * F]) + b1_ref[...]
    w1i = w1[F:2 * F]
    hi = jnp.maximum(shared + dot(ei, w1i), 0.0)
    hj = jnp.maximum(shared + dot(ej, w1i), 0.0)
    w2 = w2_ref[...]
    b2 = b2_ref[...]
    hi = jnp.maximum(dot(hi, w2) + b2, 0.0)
    hj = jnp.maximum(dot(hj, w2) + b2, 0.0)
    w3 = w3_ref[...]
    uc = eu * ec
    fmi = jnp.sum(uc * ei, axis=1, keepdims=True)
    fmj = jnp.sum(uc * ej, axis=1, keepdims=True)
    const = c_ref[...]
    oi_ref[...] = 2.0 * fmi + dot(hi, w3) + const
    oj_ref[...] = 2.0 * fmj + dot(hj, w3) + const


def _tc_fm_mlp(rows, w1t, b1r, w2t, b2r, w3t, constr):
    return pl.pallas_call(
        _tc_body,
        out_shape=[jax.ShapeDtypeStruct((B, 1), jnp.float32)] * 2,
    )(rows, w1t, b1r, w2t, b2r, w3t, constr)


def kernel(u, i, j, c, emb_table, bias_, W1, b1, W2, b2, W3, b3):
    idx = jnp.concatenate([u, i, j, c]).astype(jnp.int32).reshape(ROWS, IDX_W)
    rows = _sc_gather(emb_table, idx).reshape(NSETS, B, F)
    constr = (bias_ + b3).reshape(1, 1)
    oi, oj = _tc_fm_mlp(rows, W1.T, b1.reshape(1, -1), W2.T,
                        b2.reshape(1, -1), W3.T, constr)
    return (oi.reshape(-1), oj.reshape(-1))


# speedup vs baseline: 1.6846x; 1.6846x over previous
import functools

import jax
import jax.numpy as jnp
from jax import lax
from jax.experimental import pallas as pl
from jax.experimental.pallas import tpu as pltpu
from jax.experimental.pallas import tpu_sc as plsc

B = 4096
F = 64
NSETS = 4
IDX_W = 128
ROWS = NSETS * B // IDX_W
NC, NS = 2, 16
NW = NC * NS
RPW = ROWS // NW


NTOT = NSETS * B
BPW = NTOT // NW
UNROLL = 16


def _sc_gather(table, idx):
    mesh = plsc.VectorSubcoreMesh(core_axis_name="c", subcore_axis_name="s")

    @functools.partial(
        pl.kernel,
        mesh=mesh,
        out_type=jax.ShapeDtypeStruct((NTOT, F), jnp.float32),
        scratch_types=[
            pltpu.VMEM((BPW,), jnp.int32),
            pltpu.VMEM((BPW, F), jnp.float32),
            pltpu.SemaphoreType.DMA,
        ],
    )
    def k(table_hbm, idx_hbm, out_hbm, idx_v, rows_v, sem):
        wid = lax.axis_index("s") * NC + lax.axis_index("c")
        base = wid * BPW
        pltpu.sync_copy(idx_hbm.at[pl.ds(base, BPW)], idx_v)

        def issue(t, _):
            tb = t * UNROLL
            vec = idx_v[pl.ds(tb, UNROLL)]
            for b in range(UNROLL):
                s = vec[b]
                pltpu.async_copy(table_hbm.at[pl.ds(s, 1)],
                                 rows_v.at[pl.ds(tb + b, 1)], sem)
            return _

        lax.fori_loop(0, BPW // UNROLL, issue, None)
        pltpu.make_async_copy(table_hbm.at[pl.ds(0, BPW)], rows_v, sem).wait()
        pltpu.sync_copy(rows_v, out_hbm.at[pl.ds(base, BPW)])

    return k(table, idx)


def _tc_body(rows_ref, w1_ref, b1_ref, w2_ref, b2_ref, w3_ref, c_ref,
             oi_ref, oj_ref):
    dot = functools.partial(jax.lax.dot,
                            precision=jax.lax.Precision.HIGHEST,
                            preferred_element_type=jnp.float32)
    eu = rows_ref[0]
    ei = rows_ref[1]
    ej = rows_ref[2]
    ec = rows_ref[3]
    w1 = w1_ref[...]
    shared = dot(eu, w1[0:F]) + dot(ec, w1[2 * F:3 * F]) + b1_ref[...]
    w1i = w1[F:2 * F]
    hi = jnp.maximum(shared + dot(ei, w1i), 0.0)
    hj = jnp.maximum(shared + dot(ej, w1i), 0.0)
    w2 = w2_ref[...]
    b2 = b2_ref[...]
    hi = jnp.maximum(dot(hi, w2) + b2, 0.0)
    hj = jnp.maximum(dot(hj, w2) + b2, 0.0)
    w3 = w3_ref[...]
    uc = eu * ec
    fmi = jnp.sum(uc * ei, axis=1, keepdims=True)
    fmj = jnp.sum(uc * ej, axis=1, keepdims=True)
    const = c_ref[...]
    oi_ref[...] = 2.0 * fmi + dot(hi, w3) + const
    oj_ref[...] = 2.0 * fmj + dot(hj, w3) + const


def _tc_fm_mlp(rows, w1t, b1r, w2t, b2r, w3t, constr):
    return pl.pallas_call(
        _tc_body,
        out_shape=[jax.ShapeDtypeStruct((B, 1), jnp.float32)] * 2,
    )(rows, w1t, b1r, w2t, b2r, w3t, constr)


def kernel(u, i, j, c, emb_table, bias_, W1, b1, W2, b2, W3, b3):
    idx = jnp.concatenate([u, i, j, c]).astype(jnp.int32)
    rows = _sc_gather(emb_table, idx).reshape(NSETS, B, F)
    constr = (bias_ + b3).reshape(1, 1)
    oi, oj = _tc_fm_mlp(rows, W1.T, b1.reshape(1, -1), W2.T,
                        b2.reshape(1, -1), W3.T, constr)
    return (oi.reshape(-1), oj.reshape(-1))
